# baseline (speedup 1.0000x reference)
import jax
import jax.numpy as jnp
from jax import lax
from jax.experimental import pallas as pl
from jax.experimental.pallas import tpu as pltpu
from jax.experimental.pallas import tpu_sc as plsc

NC = 2
NS = 16
NW = NC * NS

ROWS = 16384


def _probe(b_pad: int, dim: int):
    mesh = plsc.VectorSubcoreMesh(core_axis_name="c", subcore_axis_name="s")
    half = b_pad // 2
    nrep = half // ROWS

    def body(table_hbm, idx_hbm, out_hbm, buf_sh, sem):
        sid = lax.axis_index("s")
        cid = lax.axis_index("c")

        @pl.when(sid == 0)
        def _():
            def step(i, carry):
                off = pl.multiple_of(cid * half + i * ROWS, 8)
                pltpu.async_copy(buf_sh, out_hbm.at[pl.ds(off, ROWS)], sem)
                pltpu.make_async_copy(
                    buf_sh, out_hbm.at[pl.ds(off, ROWS)], sem).wait()
                return carry

            lax.fori_loop(0, nrep, step, 0)

    return pl.kernel(
        body,
        out_type=jax.ShapeDtypeStruct((b_pad, dim), jnp.float32),
        mesh=mesh,
        scratch_types=[
            pltpu.VMEM_SHARED((ROWS, dim), jnp.float32),
            pltpu.SemaphoreType.DMA,
        ],
        compiler_params=pltpu.CompilerParams(use_tc_tiling_on_sc=False),
    )


@jax.jit
def kernel(embeddings, indices):
    n = indices.shape[0]
    dim = embeddings.shape[1]
    chunk = NW * ROWS
    b_pad = (-(-n // chunk)) * chunk
    out = _probe(b_pad, dim)(embeddings, indices)
    return out[:n]


# speedup vs baseline: 1.5596x; 1.5596x over previous
import jax
import jax.numpy as jnp
from jax import lax
from jax.experimental import pallas as pl
from jax.experimental.pallas import tpu as pltpu
from jax.experimental.pallas import tpu_sc as plsc

NC = 2
NS = 16
NW = NC * NS

BLK = 125
SBLK = 5
NBUF = 2
NIDX = 4


def _gather_grid(n: int, vocab: int, dim: int, sblocks_per_tile: int):
    mesh = plsc.VectorSubcoreMesh(core_axis_name="c", subcore_axis_name="s")
    satoms = SBLK * BLK

    def body(table_hbm, idx_hbm, out_hbm, table_sh, idx_v, rows_v, sem_idx,
             sem_gat, sem_out):
        sid = lax.axis_index("s")
        wid = sid * NC + lax.axis_index("c")
        base = wid * (sblocks_per_tile * satoms)

        @pl.when(sid == 0)
        def _():
            pltpu.sync_copy(table_hbm, table_sh)

        plsc.subcore_barrier()

        def idx_copy(s):
            blk0 = wid * (sblocks_per_tile * SBLK) + s * SBLK
            return pltpu.make_async_copy(
                idx_hbm.at[pl.ds(blk0, SBLK)], idx_v.at[s % NIDX], sem_idx)

        def gat_copy(s, j):
            return pltpu.make_async_copy(
                table_sh.at[idx_v.at[s % NIDX, j]],
                rows_v.at[s % NBUF, pl.ds(j * BLK, BLK)], sem_gat)

        def out_copy(s):
            return pltpu.make_async_copy(
                rows_v.at[s % NBUF], out_hbm.at[pl.ds(base + s * satoms,
                                                      satoms)], sem_out)

        def slot(s, retire, reclaim, prefetch):
            if retire:
                for j in range(SBLK):
                    gat_copy(s - 1, j).wait()
                out_copy(s - 1).start()
            if reclaim:
                out_copy(s - NBUF).wait()
            idx_copy(s).wait()
            for j in range(SBLK):
                gat_copy(s, j).start()
            if prefetch:
                idx_copy(s + NBUF).start()

        ns = sblocks_per_tile
        for s in range(NBUF):
            idx_copy(s).start()
        for s in range(NBUF):
            slot(s, retire=(s >= 1), reclaim=False, prefetch=(s + NBUF < ns))

        def steady(s, carry):
            slot(s, retire=True, reclaim=True, prefetch=True)
            return carry

        lax.fori_loop(NBUF, ns - NBUF, steady, 0)

        for s in range(ns - NBUF, ns):
            slot(s, retire=True, reclaim=True, prefetch=False)
        for j in range(SBLK):
            gat_copy(ns - 1, j).wait()
        out_copy(ns - 1).start()
        for s in range(ns - NBUF, ns):
            out_copy(s).wait()

    return pl.kernel(
        body,
        out_type=jax.ShapeDtypeStruct((n, dim), jnp.float32),
        mesh=mesh,
        scratch_types=[
            pltpu.VMEM_SHARED((vocab, dim), jnp.float32),
            pltpu.VMEM((NIDX, SBLK, BLK), jnp.int32),
            pltpu.VMEM((NBUF, satoms, dim), jnp.float32),
            pltpu.SemaphoreType.DMA,
            pltpu.SemaphoreType.DMA,
            pltpu.SemaphoreType.DMA,
        ],
        compiler_params=pltpu.CompilerParams(use_tc_tiling_on_sc=False),
    )


@jax.jit
def kernel(embeddings, indices):
    n = indices.shape[0]
    dim = embeddings.shape[1]
    assert n % (NW * SBLK * BLK) == 0, "index count must tile evenly"
    sblocks_per_tile = n // (NW * SBLK * BLK)
    idx2d = indices.reshape(n // BLK, BLK)
    return _gather_grid(n, embeddings.shape[0], dim,
                        sblocks_per_tile)(embeddings, idx2d)


# speedup vs baseline: 1.5614x; 1.0012x over previous
import jax
import jax.numpy as jnp
from jax import lax
from jax.experimental import pallas as pl
from jax.experimental.pallas import tpu as pltpu
from jax.experimental.pallas import tpu_sc as plsc

NC = 2
NS = 16
NW = NC * NS

BLK = 125
SBLK = 5
NBUF = 3
NIDX = 4


def _gather_grid(n: int, vocab: int, dim: int, sblocks_per_tile: int):
    mesh = plsc.VectorSubcoreMesh(core_axis_name="c", subcore_axis_name="s")
    satoms = SBLK * BLK

    def body(table_hbm, idx_hbm, out_hbm, table_sh, idx_v, rows_v, sem_idx,
             sem_gat, sem_out):
        sid = lax.axis_index("s")
        wid = sid * NC + lax.axis_index("c")
        base = wid * (sblocks_per_tile * satoms)

        @pl.when(sid == 0)
        def _():
            pltpu.sync_copy(table_hbm, table_sh)

        plsc.subcore_barrier()

        def idx_copy(s):
            blk0 = wid * (sblocks_per_tile * SBLK) + s * SBLK
            return pltpu.make_async_copy(
                idx_hbm.at[pl.ds(blk0, SBLK)], idx_v.at[s % NIDX], sem_idx)

        def gat_copy(s, j):
            return pltpu.make_async_copy(
                table_sh.at[idx_v.at[s % NIDX, j]],
                rows_v.at[s % NBUF, pl.ds(j * BLK, BLK)], sem_gat)

        def out_copy(s):
            return pltpu.make_async_copy(
                rows_v.at[s % NBUF], out_hbm.at[pl.ds(base + s * satoms,
                                                      satoms)], sem_out)

        def slot(s, retire, reclaim, prefetch):
            if retire:
                for j in range(SBLK):
                    gat_copy(s - 1, j).wait()
                out_copy(s - 1).start()
            if reclaim:
                out_copy(s - NBUF).wait()
            idx_copy(s).wait()
            for j in range(SBLK):
                gat_copy(s, j).start()
            if prefetch:
                idx_copy(s + NBUF).start()

        ns = sblocks_per_tile
        for s in range(NBUF):
            idx_copy(s).start()
        for s in range(NBUF):
            slot(s, retire=(s >= 1), reclaim=False, prefetch=(s + NBUF < ns))

        def steady(s, carry):
            slot(s, retire=True, reclaim=True, prefetch=True)
            return carry

        lax.fori_loop(NBUF, ns - NBUF, steady, 0)

        for s in range(ns - NBUF, ns):
            slot(s, retire=True, reclaim=True, prefetch=False)
        for j in range(SBLK):
            gat_copy(ns - 1, j).wait()
        out_copy(ns - 1).start()
        for s in range(ns - NBUF, ns):
            out_copy(s).wait()

    return pl.kernel(
        body,
        out_type=jax.ShapeDtypeStruct((n, dim), jnp.float32),
        mesh=mesh,
        scratch_types=[
            pltpu.VMEM_SHARED((vocab, dim), jnp.float32),
            pltpu.VMEM((NIDX, SBLK, BLK), jnp.int32),
            pltpu.VMEM((NBUF, satoms, dim), jnp.float32),
            pltpu.SemaphoreType.DMA,
            pltpu.SemaphoreType.DMA,
            pltpu.SemaphoreType.DMA,
        ],
        compiler_params=pltpu.CompilerParams(use_tc_tiling_on_sc=False),
    )


@jax.jit
def kernel(embeddings, indices):
    n = indices.shape[0]
    dim = embeddings.shape[1]
    assert n % (NW * SBLK * BLK) == 0, "index count must tile evenly"
    sblocks_per_tile = n // (NW * SBLK * BLK)
    idx2d = indices.reshape(n // BLK, BLK)
    return _gather_grid(n, embeddings.shape[0], dim,
                        sblocks_per_tile)(embeddings, idx2d)


# speedup vs baseline: 1.5821x; 1.0133x over previous
import jax
import jax.numpy as jnp
from jax import lax
from jax.experimental import pallas as pl
from jax.experimental.pallas import tpu as pltpu
from jax.experimental.pallas import tpu_sc as plsc

NC = 2
NS = 16
NW = NC * NS

BLK = 112
SBLK = 3
NBUF = 3
SPAN = 279 * BLK
TAIL = 64


def _gather_grid(n: int, vocab: int, dim: int):
    mesh = plsc.VectorSubcoreMesh(core_axis_name="c", subcore_axis_name="s")
    satoms = SBLK * BLK
    ns = SPAN // satoms
    assert NW * SPAN + TAIL == n
    fetch = SPAN + TAIL

    def body(table_hbm, idx_hbm, out_hbm, table_sh, idx_span, rows_v,
             sem_idx, sem_gat, sem_out):
        sid = lax.axis_index("s")
        wid = sid * NC + lax.axis_index("c")
        base = wid * SPAN

        idx_fetch = pltpu.make_async_copy(
            idx_hbm.at[pl.ds(pl.multiple_of(base, 8), fetch)], idx_span,
            sem_idx)
        idx_fetch.start()

        @pl.when(sid == 0)
        def _():
            pltpu.sync_copy(table_hbm, table_sh)

        idx_fetch.wait()
        plsc.subcore_barrier()

        def gat_copy(s, j):
            return pltpu.make_async_copy(
                table_sh.at[idx_span.at[pl.ds(s * satoms + j * BLK, BLK)]],
                rows_v.at[s % NBUF, pl.ds(j * BLK, BLK)], sem_gat)

        def out_copy(s):
            return pltpu.make_async_copy(
                rows_v.at[s % NBUF], out_hbm.at[pl.ds(base + s * satoms,
                                                      satoms)], sem_out)

        def slot(s, retire, reclaim):
            if retire:
                for j in range(SBLK):
                    gat_copy(s - 1, j).wait()
                out_copy(s - 1).start()
            if reclaim:
                out_copy(s - NBUF).wait()
            for j in range(SBLK):
                gat_copy(s, j).start()

        for s in range(NBUF):
            slot(s, retire=(s >= 1), reclaim=False)

        def steady(s, carry):
            slot(s, retire=True, reclaim=True)
            return carry

        lax.fori_loop(NBUF, ns, steady, 0)

        for j in range(SBLK):
            gat_copy(ns - 1, j).wait()
        out_copy(ns - 1).start()
        for s in range(ns - NBUF, ns):
            out_copy(s).wait()

        @pl.when(wid == NW - 1)
        def _():
            tail = pltpu.make_async_copy(
                table_sh.at[idx_span.at[pl.ds(SPAN, TAIL)]],
                rows_v.at[0, pl.ds(0, TAIL)], sem_gat)
            tail.start()
            tail.wait()
            pltpu.sync_copy(rows_v.at[0, pl.ds(0, TAIL)],
                            out_hbm.at[pl.ds(NW * SPAN, TAIL)])

    return pl.kernel(
        body,
        out_type=jax.ShapeDtypeStruct((n, dim), jnp.float32),
        mesh=mesh,
        scratch_types=[
            pltpu.VMEM_SHARED((vocab, dim), jnp.float32),
            pltpu.VMEM((fetch,), jnp.int32),
            pltpu.VMEM((NBUF, satoms, dim), jnp.float32),
            pltpu.SemaphoreType.DMA,
            pltpu.SemaphoreType.DMA,
            pltpu.SemaphoreType.DMA,
        ],
        compiler_params=pltpu.CompilerParams(use_tc_tiling_on_sc=False),
    )


@jax.jit
def kernel(embeddings, indices):
    n = indices.shape[0]
    dim = embeddings.shape[1]
    return _gather_grid(n, embeddings.shape[0], dim)(embeddings, indices)
